# baseline (speedup 1.0000x reference)
import functools
from math import ceil

import jax
import jax.numpy as jnp
from jax import lax
from jax.experimental import pallas as pl
from jax.experimental.pallas import tpu as pltpu
from jax.experimental.pallas import tpu_sc as plsc

N = 10000
E = 320000
D = 128
RATIO = 0.5
NEG = -jnp.inf

NC = 2
NS = 16
NW = NC * NS
CHUNK = 80
NCHUNK = E // NW // CHUNK
EPW = E // NW
NPAD = 10240
RPT = NPAD // NS
ZR = 128


@functools.partial(
    pl.kernel,
    out_type=jax.ShapeDtypeStruct((NC, NPAD, D), jnp.float32),
    mesh=plsc.VectorSubcoreMesh(core_axis_name="c", subcore_axis_name="s"),
    scratch_types=[
        pltpu.VMEM((CHUNK,), jnp.int32),
        pltpu.VMEM((CHUNK,), jnp.int32),
        pltpu.VMEM((CHUNK, D), jnp.float32),
        pltpu.VMEM((ZR, D), jnp.float32),
        pltpu.VMEM((16,), jnp.int32),
        pltpu.VMEM_SHARED((NPAD, D), jnp.float32),
        pltpu.SemaphoreType.DMA,
    ],
    compiler_params=pltpu.CompilerParams(needs_layout_passes=False),
)
def _msg_kernel(g_hbm, src_hbm, dst_hbm, cnt_hbm, zero_hbm, out_hbm,
                idx_s, idx_d, rows, bounce, cnt_v, acc, sem):
    c = lax.axis_index("c")
    s = lax.axis_index("s")
    wid = s * NC + c
    pltpu.sync_copy(cnt_hbm.at[pl.ds(wid * 16, 16)], cnt_v)

    pltpu.sync_copy(zero_hbm, bounce)
    for j in range(RPT // ZR):
        pltpu.sync_copy(bounce, acc.at[pl.ds(s * RPT + j * ZR, ZR)])
    plsc.subcore_barrier()

    def body(i, carry):
        base = wid * EPW + i * CHUNK
        pltpu.sync_copy(src_hbm.at[pl.ds(base, CHUNK)], idx_s)
        pltpu.sync_copy(dst_hbm.at[pl.ds(base, CHUNK)], idx_d)
        pltpu.async_copy(g_hbm.at[idx_s], rows, sem).wait()
        pltpu.sync_copy(rows, acc.at[idx_d], add=True)
        return carry

    nch = jnp.max(cnt_v[...])
    lax.fori_loop(0, nch, body, 0)
    plsc.subcore_barrier()

    for j in range(RPT // ZR):
        r = s * RPT + j * ZR
        pltpu.sync_copy(acc.at[pl.ds(r, ZR)], bounce)
        pltpu.sync_copy(bounce, out_hbm.at[c, pl.ds(r, ZR)])


@functools.partial(
    pl.kernel,
    out_type=[jax.ShapeDtypeStruct((E,), jnp.int32),
              jax.ShapeDtypeStruct((E,), jnp.int32),
              jax.ShapeDtypeStruct((NW * 16,), jnp.int32)],
    mesh=plsc.VectorSubcoreMesh(core_axis_name="c", subcore_axis_name="s"),
    scratch_types=[
        pltpu.VMEM((N,), jnp.float32),
        pltpu.VMEM((EPW,), jnp.int32),
        pltpu.VMEM((EPW,), jnp.int32),
        pltpu.VMEM((EPW + 96,), jnp.int32),
        pltpu.VMEM((EPW + 96,), jnp.int32),
        pltpu.VMEM((16,), jnp.int32),
    ],
    compiler_params=pltpu.CompilerParams(needs_layout_passes=False),
)
def _compact_kernel(src_hbm, dst_hbm, mask_hbm, csrc_hbm, cdst_hbm, cnt_hbm,
                    mask_v, src_v, dst_v, osrc_v, odst_v, cnt_v):
    c = lax.axis_index("c")
    s = lax.axis_index("s")
    wid = s * NC + c
    base = wid * EPW
    pltpu.sync_copy(mask_hbm, mask_v)
    pltpu.sync_copy(src_hbm.at[pl.ds(base, EPW)], src_v)
    pltpu.sync_copy(dst_hbm.at[pl.ds(base, EPW)], dst_v)

    def body(i, off):
        sv = src_v[pl.ds(i * 16, 16)]
        dv = dst_v[pl.ds(i * 16, 16)]
        ms = plsc.load_gather(mask_v, [sv])
        md = plsc.load_gather(mask_v, [dv])
        keep = (ms * md) > 0.0
        plsc.store_compressed(osrc_v.at[pl.ds(off, 16)], sv, mask=keep)
        plsc.store_compressed(odst_v.at[pl.ds(off, 16)], dv, mask=keep)
        npc = jnp.max(plsc.all_reduce_population_count(keep))
        return off + npc

    off = lax.fori_loop(0, EPW // 16, body, 0)

    zsrc = jnp.zeros((16,), jnp.int32)
    zdst = jnp.full((16,), N, jnp.int32)
    for j in range(6):
        osrc_v[pl.ds(off + j * 16, 16)] = zsrc
        odst_v[pl.ds(off + j * 16, 16)] = zdst

    nch = (off + CHUNK - 1) // CHUNK
    cnt_v[...] = jnp.full((16,), nch, jnp.int32)
    pltpu.sync_copy(cnt_v, cnt_hbm.at[pl.ds(wid * 16, 16)])
    pltpu.sync_copy(osrc_v.at[pl.ds(0, EPW)], csrc_hbm.at[pl.ds(base, EPW)])
    pltpu.sync_copy(odst_v.at[pl.ds(0, EPW)], cdst_hbm.at[pl.ds(base, EPW)])


@functools.partial(
    pl.kernel,
    out_type=jax.ShapeDtypeStruct((NW, NPAD), jnp.float32),
    mesh=plsc.VectorSubcoreMesh(core_axis_name="c", subcore_axis_name="s"),
    scratch_types=[
        pltpu.VMEM((N,), jnp.float32),
        pltpu.VMEM((EPW,), jnp.int32),
        pltpu.VMEM((EPW,), jnp.int32),
        pltpu.VMEM((NPAD,), jnp.float32),
        pltpu.VMEM((16,), jnp.int32),
    ],
    compiler_params=pltpu.CompilerParams(needs_layout_passes=False),
)
def _scalar_kernel(sn_hbm, src_hbm, dst_hbm, cnt_hbm, out_hbm,
                   sn_v, src_v, dst_v, acc_v, cnt_v):
    c = lax.axis_index("c")
    s = lax.axis_index("s")
    wid = s * NC + c
    pltpu.sync_copy(sn_hbm, sn_v)
    pltpu.sync_copy(src_hbm.at[pl.ds(wid * EPW, EPW)], src_v)
    pltpu.sync_copy(dst_hbm.at[pl.ds(wid * EPW, EPW)], dst_v)
    pltpu.sync_copy(cnt_hbm.at[pl.ds(wid * 16, 16)], cnt_v)
    zv = jnp.zeros((16,), jnp.float32)

    def zbody(i, carry):
        acc_v[pl.ds(i * 16, 16)] = zv
        return carry

    lax.fori_loop(0, NPAD // 16, zbody, 0)

    def body(i, carry):
        sidx = src_v[pl.ds(i * 16, 16)]
        v = plsc.load_gather(sn_v, [sidx])
        didx = dst_v[pl.ds(i * 16, 16)]
        plsc.addupdate_scatter(acc_v, [didx], v)
        return carry

    nch = jnp.max(cnt_v[...])
    lax.fori_loop(0, nch * (CHUNK // 16), body, 0)
    pltpu.sync_copy(acc_v, out_hbm.at[wid])


RB = 1000
GRID = N // RB


def _mm_body(x_ref, w_ref, o_ref):
    o_ref[...] = jnp.dot(x_ref[...], w_ref[...], preferred_element_type=jnp.float32)


def _mm(x, w):
    return pl.pallas_call(
        _mm_body,
        grid=(GRID,),
        in_specs=[pl.BlockSpec((RB, x.shape[1]), lambda i: (i, 0)),
                  pl.BlockSpec((w.shape[0], w.shape[1]), lambda i: (0, 0))],
        out_specs=pl.BlockSpec((RB, w.shape[1]), lambda i: (i, 0)),
        out_shape=jax.ShapeDtypeStruct((x.shape[0], w.shape[1]), jnp.float32),
    )(x, w)


def _dense_body(p_ref, hr_ref, cb_ref, m_ref, w_ref, h_ref, hw_ref):
    h = hr_ref[...] + p_ref[0] + p_ref[1] + cb_ref[...]
    h = jax.nn.relu(h) * m_ref[...]
    h_ref[...] = h
    hw_ref[...] = jnp.dot(h, w_ref[...], preferred_element_type=jnp.float32)


def _dense(parts, hr, cb, mask_col, wcat):
    return pl.pallas_call(
        _dense_body,
        grid=(GRID,),
        in_specs=[pl.BlockSpec((NC, RB, D), lambda i: (0, i, 0)),
                  pl.BlockSpec((RB, D), lambda i: (i, 0)),
                  pl.BlockSpec((1, D), lambda i: (0, 0)),
                  pl.BlockSpec((RB, 1), lambda i: (i, 0)),
                  pl.BlockSpec((D, 384), lambda i: (0, 0))],
        out_specs=[pl.BlockSpec((RB, D), lambda i: (i, 0)),
                   pl.BlockSpec((RB, 384), lambda i: (i, 0))],
        out_shape=[jax.ShapeDtypeStruct((N, D), jnp.float32),
                   jax.ShapeDtypeStruct((N, 384), jnp.float32)],
    )(parts, hr, cb, mask_col, wcat)


def _thresh_body(k, sr_ref, sp_ref, m_ref, pb_ref, mn_ref, t_ref):
    sagg = jnp.sum(sp_ref[...], axis=0).reshape(NPAD // 128, 128)
    s = sr_ref[...] + sagg + pb_ref[0, 0]
    sm = jnp.where(m_ref[...] > 0, s, NEG)
    b = jax.lax.bitcast_convert_type(sm, jnp.uint32)
    ukey = jnp.where(b >= jnp.uint32(0x80000000), ~b, b | jnp.uint32(0x80000000))

    def bit_body(i, thr):
        bit = jnp.left_shift(jnp.uint32(1), jnp.uint32(31) - i.astype(jnp.uint32))
        cand = thr | bit
        cnt = jnp.sum((ukey >= cand).astype(jnp.int32))
        return jnp.where(cnt >= k, cand, thr)

    thr = lax.fori_loop(0, 32, bit_body, jnp.uint32(0))
    mn = (ukey >= thr).astype(jnp.float32)
    mn_ref[...] = mn
    t_ref[...] = jnp.tanh(sm) * mn


def _thresh(k, sr_pad, sparts, mask_pad, pb):
    return pl.pallas_call(
        functools.partial(_thresh_body, k),
        out_shape=[jax.ShapeDtypeStruct((NPAD // 128, 128), jnp.float32),
                   jax.ShapeDtypeStruct((NPAD // 128, 128), jnp.float32)],
    )(sr_pad, sparts, mask_pad, pb)


def _scale_body(h_ref, g_ref, r_ref, t_ref, m_ref, go_ref, ho_ref, rmax_ref, rsum_ref):
    i = pl.program_id(0)
    t = t_ref[...]
    hm = h_ref[...] * t
    go_ref[...] = g_ref[...] * t
    ho_ref[...] = r_ref[...] * t
    bmax = jnp.max(jnp.where(m_ref[...] > 0, hm, NEG), axis=0, keepdims=True)
    bsum = jnp.sum(hm, axis=0, keepdims=True)

    @pl.when(i == 0)
    def _():
        rmax_ref[...] = bmax
        rsum_ref[...] = bsum

    @pl.when(i > 0)
    def _():
        rmax_ref[...] = jnp.maximum(rmax_ref[...], bmax)
        rsum_ref[...] = rsum_ref[...] + bsum


def _scale_readout(h, g, r, t_col, mask_col):
    return pl.pallas_call(
        _scale_body,
        grid=(GRID,),
        in_specs=[pl.BlockSpec((RB, D), lambda i: (i, 0)),
                  pl.BlockSpec((RB, D), lambda i: (i, 0)),
                  pl.BlockSpec((RB, D), lambda i: (i, 0)),
                  pl.BlockSpec((RB, 1), lambda i: (i, 0)),
                  pl.BlockSpec((RB, 1), lambda i: (i, 0))],
        out_specs=[pl.BlockSpec((RB, D), lambda i: (i, 0)),
                   pl.BlockSpec((RB, D), lambda i: (i, 0)),
                   pl.BlockSpec((1, D), lambda i: (0, 0)),
                   pl.BlockSpec((1, D), lambda i: (0, 0))],
        out_shape=[jax.ShapeDtypeStruct((N, D), jnp.float32),
                   jax.ShapeDtypeStruct((N, D), jnp.float32),
                   jax.ShapeDtypeStruct((1, D), jnp.float32),
                   jax.ShapeDtypeStruct((1, D), jnp.float32)],
    )(h, g, r, t_col, mask_col)


def _mlp_body(r1_ref, s1_ref, r2_ref, s2_ref, r3_ref, s3_ref,
              w1_ref, b1_ref, w2_ref, b2_ref, w3_ref, b3_ref, out_ref):
    k1, k2, k3 = 5000, 2500, 1250
    z = jnp.concatenate(
        [r1_ref[...] + r2_ref[...] + r3_ref[...],
         s1_ref[...] / k1 + s2_ref[...] / k2 + s3_ref[...] / k3], axis=1)
    z = jax.nn.relu(z @ w1_ref[...] + b1_ref[...][None, :])
    z = jax.nn.relu(z @ w2_ref[...] + b2_ref[...][None, :])
    z = z @ w3_ref[...] + b3_ref[...][None, :]
    m = jnp.max(z, axis=-1, keepdims=True)
    e = jnp.exp(z - m)
    out_ref[...] = z - m - jnp.log(jnp.sum(e, axis=-1, keepdims=True))


def _mlp(ro, w1, b1, w2, b2, w3, b3):
    return pl.pallas_call(
        _mlp_body,
        out_shape=jax.ShapeDtypeStruct((1, w3.shape[1]), jnp.float32),
    )(ro[0], ro[1], ro[2], ro[3], ro[4], ro[5], w1, b1, w2, b2, w3, b3)


def kernel(x, c1_wr, c1_wn, c1_b, p1_wr, p1_wn, p1_b,
           c2_wr, c2_wn, c2_b, p2_wr, p2_wn, p2_b,
           c3_wr, c3_wn, c3_b, p3_wr, p3_wn, p3_b,
           l1_w, l1_b, l2_w, l2_b, l3_w, l3_b,
           edge_index, two_hop, batch):
    src, dst = edge_index[0], edge_index[1]
    zeros = jnp.zeros((ZR, D), jnp.float32)
    cnt = jnp.full((NW * 16,), NCHUNK, jnp.int32)
    wpad = jnp.zeros((D, 384 - 2 * D - 2), jnp.float32)
    wz = jnp.zeros((D, D), jnp.float32)
    mm0 = _mm(x, jnp.concatenate([c1_wr, c1_wn], axis=1))
    hr, g = mm0[:, :D], mm0[:, D:]
    mask_col = jnp.ones((N, 1), jnp.float32)
    mask_pad = jnp.concatenate([jnp.ones((N,), jnp.float32),
                                jnp.zeros((NPAD - N,), jnp.float32)]
                               ).reshape(NPAD // 128, 128)
    k_prev = N
    ro = []
    layers = [(c1_b, p1_wr, p1_wn, p1_b, c2_wn, c2_wr),
              (c2_b, p2_wr, p2_wn, p2_b, c3_wn, c3_wr),
              (c3_b, p3_wr, p3_wn, p3_b, wz, wz)]
    for li, (cb, pwr, pwn, pb, wn_next, wr_next) in enumerate(layers):
        wcat = jnp.concatenate([wn_next, wr_next, pwr, pwn, wpad], axis=1)
        parts = _msg_kernel(g, src, dst, cnt, zeros)
        h, hw = _dense(parts, hr, cb.reshape(1, D), mask_col, wcat)
        sn = hw[:, 2 * D + 1]
        sparts = _scalar_kernel(sn, src, dst, cnt)
        sr_pad = jnp.pad(hw[:, 2 * D], (0, NPAD - N)).reshape(NPAD // 128, 128)
        k = int(ceil(RATIO * k_prev))
        mn80, t80 = _thresh(k, sr_pad, sparts, mask_pad, pb.reshape(1, 1))
        mn_flat = mn80.reshape(-1)[:N]
        t_col = t80.reshape(-1)[:N, None]
        mask_col = mn_flat[:, None]
        g, hr, rmax, rsum = _scale_readout(h, hw[:, :D], hw[:, D:2 * D],
                                           t_col, mask_col)
        ro += [rmax, rsum]
        if li < 2:
            src, dst, cnt = _compact_kernel(edge_index[0], edge_index[1], mn_flat)
        mask_pad = mn80
        k_prev = k
    return _mlp(ro, l1_w, l1_b, l2_w, l2_b, l3_w, l3_b)
